# baseline (speedup 1.0000x reference)
import functools

import jax
import jax.numpy as jnp
from jax import lax
from jax.experimental import pallas as pl
from jax.experimental.pallas import tpu as pltpu
from jax.experimental.pallas import tpu_sc as plsc

_NC = 2
_NS = 16
_L = 16
_K = 4


@functools.cache
def _build(B: int, D: int):
    NW = _NC * _NS
    assert B % (NW * _L) == 0 and D == 2 * _L
    b_per_w = B // NW
    nb = b_per_w // _K
    mesh = plsc.VectorSubcoreMesh(core_axis_name="c", subcore_axis_name="s")

    @functools.partial(
        pl.kernel,
        out_type=jax.ShapeDtypeStruct((B,), jnp.float32),
        mesh=mesh,
        compiler_params=pltpu.CompilerParams(use_tc_tiling_on_sc=True,
                                             needs_layout_passes=False),
        scratch_types=[
            pltpu.VMEM((b_per_w,), jnp.int32),
            pltpu.VMEM((b_per_w,), jnp.int32),
            pltpu.VMEM((2 * _K, D, 128), jnp.float32),
            pltpu.VMEM((2 * _K, D, 128), jnp.float32),
            pltpu.VMEM((b_per_w,), jnp.float32),
            pltpu.SemaphoreType.DMA,
            pltpu.SemaphoreType.DMA,
        ],
    )
    def k(uidx_hbm, iidx_hbm, utt_hbm, itt_hbm, out_hbm,
          uidx_v, iidx_v, ublk, iblk, out_v, sem0, sem1):
        sems = (sem0, sem1)
        wid = lax.axis_index("s") * _NC + lax.axis_index("c")
        base = wid * b_per_w
        pltpu.sync_copy(uidx_hbm.at[pl.ds(base, b_per_w)], uidx_v)
        pltpu.sync_copy(iidx_hbm.at[pl.ds(base, b_per_w)], iidx_v)

        lanes = lax.iota(jnp.int32, _L)
        hi = lanes + jnp.full((_L,), _L, jnp.int32)
        zero_i = jnp.zeros((_L,), jnp.int32)

        def batch_vecs(b):
            vu = uidx_v[pl.ds(lax.div(b, 4) * _L, _L)]
            vi = iidx_v[pl.ds(lax.div(b, 4) * _L, _L)]
            loff = lax.rem(b, 4) * _K
            return vu, vi, loff

        def extract(vec, lane_id):
            return jnp.sum(jnp.where(lanes == lane_id, vec, zero_i))

        def issue(b, parity):
            vu, vi, loff = batch_vecs(b)
            rbu = lax.shift_right_logical(vu, 7)
            rbi = lax.shift_right_logical(vi, 7)
            for l in range(_K):
                slot = parity * _K + l
                lane_id = jnp.full((_L,), loff + l, jnp.int32)
                cu = pl.multiple_of(extract(rbu, lane_id) * 128, 128)
                ci = pl.multiple_of(extract(rbi, lane_id) * 128, 128)
                for q in range(D // 8):
                    qs = pl.ds(q * 8, 8)
                    pltpu.async_copy(utt_hbm.at[qs, pl.ds(cu, 128)],
                                     ublk.at[slot, qs], sems[parity])
                    pltpu.async_copy(itt_hbm.at[qs, pl.ds(ci, 128)],
                                     iblk.at[slot, qs], sems[parity])

        def body(b, res):
            for parity in range(2):
                @pl.when((b < nb) & (lax.rem(b, 2) == parity))
                def _(parity=parity):
                    issue(b, parity)

            @pl.when(b >= 1)
            def _():
                for parity in range(2):
                    @pl.when(lax.rem(b - 1, 2) == parity)
                    def _(parity=parity):
                        for _ in range(2 * _K):
                            pltpu.make_async_copy(
                                utt_hbm.at[:, pl.ds(0, 128)], ublk.at[0],
                                sems[parity]).wait()

            def compute(res):
                vu, vi, loff = batch_vecs(b - 1)
                for l in range(_K):
                    lane_id = jnp.full((_L,), loff + l, jnp.int32)
                    ru = extract(vu, lane_id)
                    ri = extract(vi, lane_id)
                    mu = jnp.full((_L,), ru & 127, jnp.int32)
                    mi = jnp.full((_L,), ri & 127, jnp.int32)
                    pbase = lax.rem(b - 1, 2) * _K + l
                    kf = jnp.full((_L,), pbase, jnp.int32)
                    u_lo = plsc.load_gather(ublk, [kf, lanes, mu])
                    u_hi = plsc.load_gather(ublk, [kf, hi, mu])
                    v_lo = plsc.load_gather(iblk, [kf, lanes, mi])
                    v_hi = plsc.load_gather(iblk, [kf, hi, mi])
                    dot = jnp.sum(u_lo * v_lo + u_hi * v_hi)
                    res = jnp.where(lanes == lane_id, dot, res)
                return res

            def flush(res):
                out_v[pl.ds(lax.div(b - 1, 4) * _L, _L)] = res
                return res

            res = lax.cond(b >= 1, compute, lambda r: r, res)
            res = lax.cond((b >= 1) & (lax.rem(b, 4) == 0), flush,
                           lambda r: r, res)
            return res

        lax.fori_loop(0, nb + 1, body, jnp.zeros((_L,), jnp.float32))
        pltpu.sync_copy(out_v, out_hbm.at[pl.ds(base, b_per_w)])

    return k


def kernel(user_item_tuple, user_table, item_table):
    uid = user_item_tuple[:, 0].astype(jnp.int32)
    iid = user_item_tuple[:, 1].astype(jnp.int32)
    n, d = user_table.shape
    return _build(uid.shape[0], d)(uid, iid, user_table.T, item_table.T)


# speedup vs baseline: 1.0078x; 1.0078x over previous
import functools

import jax
import jax.numpy as jnp
from jax import lax
from jax.experimental import pallas as pl
from jax.experimental.pallas import tpu as pltpu
from jax.experimental.pallas import tpu_sc as plsc

_NC = 2
_NS = 16
_L = 16
_K = 4


@functools.cache
def _build(B: int, D: int):
    NW = _NC * _NS
    assert B % (NW * _L) == 0 and D == 2 * _L
    b_per_w = B // NW
    nb = b_per_w // _K
    mesh = plsc.VectorSubcoreMesh(core_axis_name="c", subcore_axis_name="s")

    @functools.partial(
        pl.kernel,
        out_type=jax.ShapeDtypeStruct((B,), jnp.float32),
        mesh=mesh,
        compiler_params=pltpu.CompilerParams(use_tc_tiling_on_sc=True,
                                             needs_layout_passes=False),
        scratch_types=[
            pltpu.VMEM((b_per_w,), jnp.int32),
            pltpu.VMEM((b_per_w,), jnp.int32),
            pltpu.VMEM((2 * _K, D, 128), jnp.float32),
            pltpu.VMEM((2 * _K, D, 128), jnp.float32),
            pltpu.VMEM((b_per_w,), jnp.float32),
            pltpu.SemaphoreType.DMA,
            pltpu.SemaphoreType.DMA,
        ],
    )
    def k(uidx_hbm, iidx_hbm, utt_hbm, itt_hbm, out_hbm,
          uidx_v, iidx_v, ublk, iblk, out_v, sem0, sem1):
        sems = (sem0, sem1)
        wid = lax.axis_index("s") * _NC + lax.axis_index("c")
        base = wid * b_per_w
        pltpu.sync_copy(uidx_hbm.at[pl.ds(base, b_per_w)], uidx_v)
        pltpu.sync_copy(iidx_hbm.at[pl.ds(base, b_per_w)], iidx_v)

        lanes = lax.iota(jnp.int32, _L)
        hi = lanes + jnp.full((_L,), _L, jnp.int32)
        zero_i = jnp.zeros((_L,), jnp.int32)

        def batch_vecs(b):
            vu = uidx_v[pl.ds(lax.div(b, 4) * _L, _L)]
            vi = iidx_v[pl.ds(lax.div(b, 4) * _L, _L)]
            loff = lax.rem(b, 4) * _K
            return vu, vi, loff

        def extract(vec, lane_id):
            return jnp.sum(jnp.where(lanes == lane_id, vec, zero_i))

        def issue(b, parity):
            vu, vi, loff = batch_vecs(b)
            rbu = lax.shift_right_logical(vu, 7)
            rbi = lax.shift_right_logical(vi, 7)
            for l in range(_K):
                slot = parity * _K + l
                lane_id = jnp.full((_L,), loff + l, jnp.int32)
                cu = pl.multiple_of(extract(rbu, lane_id) * 128, 128)
                ci = pl.multiple_of(extract(rbi, lane_id) * 128, 128)
                pltpu.async_copy(utt_hbm.at[:, pl.ds(cu, 128)],
                                 ublk.at[slot], sems[parity])
                pltpu.async_copy(itt_hbm.at[:, pl.ds(ci, 128)],
                                 iblk.at[slot], sems[parity])

        def body(b, res):
            for parity in range(2):
                @pl.when((b < nb) & (lax.rem(b, 2) == parity))
                def _(parity=parity):
                    issue(b, parity)

            @pl.when(b >= 1)
            def _():
                for parity in range(2):
                    @pl.when(lax.rem(b - 1, 2) == parity)
                    def _(parity=parity):
                        for _ in range(2 * _K):
                            pltpu.make_async_copy(
                                utt_hbm.at[:, pl.ds(0, 128)], ublk.at[0],
                                sems[parity]).wait()

            def compute(res):
                vu, vi, loff = batch_vecs(b - 1)
                for l in range(_K):
                    lane_id = jnp.full((_L,), loff + l, jnp.int32)
                    ru = extract(vu, lane_id)
                    ri = extract(vi, lane_id)
                    mu = jnp.full((_L,), ru & 127, jnp.int32)
                    mi = jnp.full((_L,), ri & 127, jnp.int32)
                    pbase = lax.rem(b - 1, 2) * _K + l
                    kf = jnp.full((_L,), pbase, jnp.int32)
                    u_lo = plsc.load_gather(ublk, [kf, lanes, mu])
                    u_hi = plsc.load_gather(ublk, [kf, hi, mu])
                    v_lo = plsc.load_gather(iblk, [kf, lanes, mi])
                    v_hi = plsc.load_gather(iblk, [kf, hi, mi])
                    dot = jnp.sum(u_lo * v_lo + u_hi * v_hi)
                    res = jnp.where(lanes == lane_id, dot, res)
                return res

            def flush(res):
                out_v[pl.ds(lax.div(b - 1, 4) * _L, _L)] = res
                return res

            res = lax.cond(b >= 1, compute, lambda r: r, res)
            res = lax.cond((b >= 1) & (lax.rem(b, 4) == 0), flush,
                           lambda r: r, res)
            return res

        lax.fori_loop(0, nb + 1, body, jnp.zeros((_L,), jnp.float32))
        pltpu.sync_copy(out_v, out_hbm.at[pl.ds(base, b_per_w)])

    return k


def kernel(user_item_tuple, user_table, item_table):
    uid = user_item_tuple[:, 0].astype(jnp.int32)
    iid = user_item_tuple[:, 1].astype(jnp.int32)
    n, d = user_table.shape
    return _build(uid.shape[0], d)(uid, iid, user_table.T, item_table.T)
